# baseline (speedup 1.0000x reference)
import functools

import jax
import jax.numpy as jnp
from jax import lax
from jax.experimental import pallas as pl
from jax.experimental.pallas import tpu as pltpu
from jax.experimental.pallas import tpu_sc as plsc

N_VOCAB = 100000
N_DEC = N_VOCAB - 1
MAX_PATH = 32
D = 512
B = 8192

NC = 2
NS = 16
LANES = 16
NW = NC * NS
TPW = B // NW
NCHUNK = D // LANES
AUXW = 128
SENTINEL = 1e4
SEC = 64


def _sc_scores():
    mesh = plsc.VectorSubcoreMesh(core_axis_name="c", subcore_axis_name="s")

    @functools.partial(
        pl.kernel,
        out_type=jax.ShapeDtypeStruct((B, MAX_PATH), jnp.float32),
        mesh=mesh,
        compiler_params=pltpu.CompilerParams(needs_layout_passes=False),
        scratch_types=[
            pltpu.VMEM((TPW,), jnp.int32),
            pltpu.VMEM((2, SEC, AUXW), jnp.int32),
            pltpu.VMEM((4, 1, D), jnp.float32),
            pltpu.VMEM((4, MAX_PATH, D), jnp.float32),
            pltpu.VMEM((LANES * LANES,), jnp.float32),
            pltpu.VMEM((TPW, MAX_PATH), jnp.float32),
            pltpu.SemaphoreType.DMA((2,)),
            [pltpu.SemaphoreType.DMA] * 4,
            [pltpu.SemaphoreType.DMA] * 4,
            [pltpu.SemaphoreType.DMA] * 4,
        ],
    )
    def sc_kernel(x_hbm, t_hbm, aux_hbm, decs_hbm,
                  scores_hbm,
                  t_v, aux_v, xrow_v, rows_v, accs_v, sb_v,
                  sema, semr, semh, semx):
        wid = lax.axis_index("s") * NC + lax.axis_index("c")
        base = wid * TPW

        pltpu.sync_copy(t_hbm.at[pl.ds(base, TPW)], t_v)

        iota16 = lax.iota(jnp.int32, LANES)
        sent16 = jnp.full((LANES,), SENTINEL, jnp.float32)

        def secfire(s):
            pltpu.async_copy(aux_hbm.at[t_v.at[pl.ds(s * SEC, SEC)]],
                             aux_v.at[s % 2], sema.at[s % 2])

        def secwait(s):
            pltpu.make_async_copy(aux_hbm.at[t_v.at[pl.ds(s * SEC, SEC)]],
                                  aux_v.at[s % 2], sema.at[s % 2]).wait()

        def lenvec(i):
            return aux_v[(i // SEC) % 2, i % SEC, pl.ds(MAX_PATH, LANES)]

        def auxrow(i, sl):
            return aux_v.at[(i // SEC) % 2, i % SEC, sl]

        def fire(i, slot):
            pltpu.async_copy(decs_hbm.at[auxrow(i, pl.ds(0, LANES))],
                             rows_v.at[slot, pl.ds(0, LANES)], semr[slot])

            @pl.when(lenvec(i)[0] > LANES)
            def _():
                pltpu.async_copy(decs_hbm.at[auxrow(i, pl.ds(LANES, LANES))],
                                 rows_v.at[slot, pl.ds(LANES, LANES)],
                                 semh[slot])

            pltpu.async_copy(x_hbm.at[pl.ds(base + i, 1)],
                             xrow_v.at[slot], semx[slot])

        def wait(i, slot):
            pltpu.make_async_copy(decs_hbm.at[auxrow(i, pl.ds(0, LANES))],
                                  rows_v.at[slot, pl.ds(0, LANES)],
                                  semr[slot]).wait()
            pltpu.make_async_copy(x_hbm.at[pl.ds(base + i, 1)],
                                  xrow_v.at[slot], semx[slot]).wait()

            @pl.when(lenvec(i)[0] > LANES)
            def _():
                pltpu.make_async_copy(
                    decs_hbm.at[auxrow(i, pl.ds(LANES, LANES))],
                    rows_v.at[slot, pl.ds(LANES, LANES)], semh[slot]).wait()

        def group(i, slot, jg):
            rows = rows_v.at[slot]
            len_bc = lenvec(i)
            zero = jnp.zeros((LANES,), jnp.float32)

            def cb_body(cb, accs):
                out = list(accs)
                for u in range(4):
                    off = (cb * 4 + u) * LANES
                    xc = xrow_v[slot, 0, pl.ds(off, LANES)]
                    for j16 in range(LANES):
                        j = jg * LANES + j16
                        out[j16] = out[j16] + rows[j, pl.ds(off, LANES)] * xc
                return tuple(out)

            accs = lax.fori_loop(0, NCHUNK // 4, cb_body, (zero,) * LANES)
            for j16 in range(LANES):
                accs_v[pl.ds(j16 * LANES, LANES)] = accs[j16]
            row_base = iota16 * LANES
            s0 = plsc.load_gather(accs_v, [row_base])
            s1 = plsc.load_gather(accs_v, [row_base + 1])
            for k in range(2, LANES, 2):
                s0 = s0 + plsc.load_gather(accs_v, [row_base + k])
                s1 = s1 + plsc.load_gather(accs_v, [row_base + k + 1])
            mask = (iota16 + jg * LANES) < len_bc
            s = jnp.where(mask, s0 + s1, SENTINEL)
            sb_v[i, pl.ds(jg * LANES, LANES)] = s

        def compute(i, slot):
            group(i, slot, 0)
            sb_v[i, pl.ds(LANES, LANES)] = sent16

            @pl.when(lenvec(i)[0] > LANES)
            def _():
                group(i, slot, 1)

        secfire(0)
        secwait(0)
        secfire(1)
        for s in range(3):
            fire(s, s)

        QPS = SEC // 4

        def quad_body(g, _):
            i0 = 4 * g

            @pl.when((g % QPS == 0) & (g > 0) & (g // QPS + 1 < TPW // SEC))
            def _():
                secfire(g // QPS + 1)

            @pl.when((g % QPS == QPS - 1) & (g // QPS + 1 < TPW // SEC))
            def _():
                secwait(g // QPS + 1)

            for u in range(4):
                i = i0 + u

                @pl.when(i + 3 < TPW)
                def _():
                    fire(i + 3, (u + 3) % 4)

                wait(i, u)
                compute(i, u)
            return 0

        lax.fori_loop(0, TPW // 4, quad_body, 0)
        pltpu.sync_copy(sb_v, scores_hbm.at[pl.ds(base, TPW)])

    return sc_kernel


_SC_SCORES = _sc_scores()


_AUX_BK = 2000


def _aux_body(p_ref, l_ref, o_ref):
    p = p_ref[...]
    l = l_ref[0]
    lt = jnp.transpose(jnp.broadcast_to(l, (LANES, _AUX_BK)), (1, 0))
    o_ref[...] = jnp.concatenate(
        [
            p,
            lt,
            jnp.zeros((_AUX_BK, AUXW - MAX_PATH - LANES), jnp.int32),
        ],
        axis=1,
    )


_AUX_BUILD = pl.pallas_call(
    _aux_body,
    grid=(N_VOCAB // _AUX_BK,),
    in_specs=[
        pl.BlockSpec((_AUX_BK, MAX_PATH), lambda i: (i, 0)),
        pl.BlockSpec((1, 1, _AUX_BK), lambda i: (i, 0, 0)),
    ],
    out_specs=pl.BlockSpec((_AUX_BK, AUXW), lambda i: (i, 0)),
    out_shape=jax.ShapeDtypeStruct((N_VOCAB, AUXW), jnp.int32),
)


def _tc_loss_body(s_ref, o_ref):
    ls = jax.nn.log_sigmoid(s_ref[...])
    o_ref[...] = jnp.reshape(-jnp.sum(ls) / B, (1, 1))


_TC_LOSS = pl.pallas_call(
    _tc_loss_body,
    out_shape=jax.ShapeDtypeStruct((1, 1), jnp.float32),
)


def kernel(x, t, decs, paths, lens):
    t = t.astype(jnp.int32)
    aux = _AUX_BUILD(paths.astype(jnp.int32),
                     lens.astype(jnp.int32).reshape(N_VOCAB // _AUX_BK, 1,
                                                    _AUX_BK))
    scores = _SC_SCORES(x, t, aux, decs)
    loss = _TC_LOSS(scores)
    return loss[0, 0]


# speedup vs baseline: 1.0434x; 1.0434x over previous
import functools

import jax
import jax.numpy as jnp
from jax import lax
from jax.experimental import pallas as pl
from jax.experimental.pallas import tpu as pltpu
from jax.experimental.pallas import tpu_sc as plsc

N_VOCAB = 100000
N_DEC = N_VOCAB - 1
MAX_PATH = 32
D = 512
B = 8192

NC = 2
NS = 16
LANES = 16
NW = NC * NS
TPW = B // NW
NCHUNK = D // LANES
AUXW = 128
SENTINEL = 1e4


def _sc_scores():
    mesh = plsc.VectorSubcoreMesh(core_axis_name="c", subcore_axis_name="s")

    @functools.partial(
        pl.kernel,
        out_type=jax.ShapeDtypeStruct((B, MAX_PATH), jnp.float32),
        mesh=mesh,
        compiler_params=pltpu.CompilerParams(needs_layout_passes=False),
        scratch_types=[
            pltpu.VMEM((TPW,), jnp.int32),
            pltpu.VMEM((TPW, AUXW), jnp.int32),
            pltpu.VMEM((2, 16, D), jnp.float32),
            pltpu.VMEM((2, MAX_PATH, D), jnp.float32),
            pltpu.VMEM((LANES * LANES,), jnp.float32),
            pltpu.VMEM((TPW, MAX_PATH), jnp.float32),
            pltpu.SemaphoreType.DMA,
            pltpu.SemaphoreType.DMA,
            pltpu.SemaphoreType.DMA,
            pltpu.SemaphoreType.DMA,
            pltpu.SemaphoreType.DMA,
            pltpu.SemaphoreType.DMA,
        ],
    )
    def sc_kernel(x_hbm, t_hbm, aux_hbm, decs_hbm,
                  scores_hbm,
                  t_v, aux_v, xrow_v, rows_v, accs_v, sb_v,
                  semr0, semr1, semh0, semh1, semx0, semx1):
        wid = lax.axis_index("s") * NC + lax.axis_index("c")
        base = wid * TPW

        pltpu.sync_copy(t_hbm.at[pl.ds(base, TPW)], t_v)
        for g in range(TPW // 128):
            sl = pl.ds(g * 128, 128)
            pltpu.async_copy(aux_hbm.at[t_v.at[sl]], aux_v.at[sl],
                             semr0).wait()

        iota16 = lax.iota(jnp.int32, LANES)
        sent16 = jnp.full((LANES,), SENTINEL, jnp.float32)
        semr = (semr0, semr1)
        semh = (semh0, semh1)
        semx = (semx0, semx1)

        def lenvec(i):
            return aux_v[i, pl.ds(MAX_PATH, LANES)]

        def fire(i, slot):
            pltpu.async_copy(decs_hbm.at[aux_v.at[i, pl.ds(0, LANES)]],
                             rows_v.at[slot, pl.ds(0, LANES)], semr[slot])

            @pl.when(lenvec(i)[0] > LANES)
            def _():
                pltpu.async_copy(decs_hbm.at[aux_v.at[i, pl.ds(LANES, LANES)]],
                                 rows_v.at[slot, pl.ds(LANES, LANES)],
                                 semh[slot])

        def wait(i, slot):
            pltpu.make_async_copy(decs_hbm.at[aux_v.at[i, pl.ds(0, LANES)]],
                                  rows_v.at[slot, pl.ds(0, LANES)],
                                  semr[slot]).wait()

            @pl.when(lenvec(i)[0] > LANES)
            def _():
                pltpu.make_async_copy(
                    decs_hbm.at[aux_v.at[i, pl.ds(LANES, LANES)]],
                    rows_v.at[slot, pl.ds(LANES, LANES)], semh[slot]).wait()

        def group(i, slot, jg):
            rows = rows_v.at[slot]
            len_bc = lenvec(i)
            zero = jnp.zeros((LANES,), jnp.float32)

            def cb_body(cb, accs):
                out = list(accs)
                for u in range(4):
                    off = (cb * 4 + u) * LANES
                    xc = xrow_v[(i // 16) % 2, i % 16, pl.ds(off, LANES)]
                    for j16 in range(LANES):
                        j = jg * LANES + j16
                        out[j16] = out[j16] + rows[j, pl.ds(off, LANES)] * xc
                return tuple(out)

            accs = lax.fori_loop(0, NCHUNK // 4, cb_body, (zero,) * LANES)
            for j16 in range(LANES):
                accs_v[pl.ds(j16 * LANES, LANES)] = accs[j16]
            row_base = iota16 * LANES
            s0 = plsc.load_gather(accs_v, [row_base])
            s1 = plsc.load_gather(accs_v, [row_base + 1])
            for k in range(2, LANES, 2):
                s0 = s0 + plsc.load_gather(accs_v, [row_base + k])
                s1 = s1 + plsc.load_gather(accs_v, [row_base + k + 1])
            mask = (iota16 + jg * LANES) < len_bc
            s = jnp.where(mask, s0 + s1, SENTINEL)
            sb_v[i, pl.ds(jg * LANES, LANES)] = s

        def compute(i, slot):
            group(i, slot, 0)
            sb_v[i, pl.ds(LANES, LANES)] = sent16

            @pl.when(lenvec(i)[0] > LANES)
            def _():
                group(i, slot, 1)

        def xfire(c):
            pltpu.async_copy(x_hbm.at[pl.ds(base + c * 16, 16)],
                             xrow_v.at[c % 2], semx0)

        def xwait(c):
            pltpu.make_async_copy(x_hbm.at[pl.ds(base + c * 16, 16)],
                                  xrow_v.at[c % 2], semx0).wait()

        xfire(0)
        xwait(0)
        fire(0, 0)

        def pair_body(g, _):
            i0 = 2 * g

            @pl.when((g % 8 == 4) & (i0 // 16 + 1 < TPW // 16))
            def _():
                xfire(i0 // 16 + 1)

            @pl.when((g % 8 == 0) & (g > 0))
            def _():
                xwait(i0 // 16)

            fire(i0 + 1, 1)
            wait(i0, 0)
            compute(i0, 0)

            @pl.when(g < TPW // 2 - 1)
            def _():
                fire(i0 + 2, 0)

            wait(i0 + 1, 1)
            compute(i0 + 1, 1)
            return 0

        lax.fori_loop(0, TPW // 2, pair_body, 0)
        pltpu.sync_copy(sb_v, scores_hbm.at[pl.ds(base, TPW)])

    return sc_kernel


_SC_SCORES = _sc_scores()


_AUX_BK = 2000


def _aux_body(p_ref, l_ref, o_ref):
    p = p_ref[...]
    l = l_ref[0]
    lt = jnp.transpose(jnp.broadcast_to(l, (LANES, _AUX_BK)), (1, 0))
    o_ref[...] = jnp.concatenate(
        [
            p,
            lt,
            jnp.zeros((_AUX_BK, AUXW - MAX_PATH - LANES), jnp.int32),
        ],
        axis=1,
    )


_AUX_BUILD = pl.pallas_call(
    _aux_body,
    grid=(N_VOCAB // _AUX_BK,),
    in_specs=[
        pl.BlockSpec((_AUX_BK, MAX_PATH), lambda i: (i, 0)),
        pl.BlockSpec((1, 1, _AUX_BK), lambda i: (i, 0, 0)),
    ],
    out_specs=pl.BlockSpec((_AUX_BK, AUXW), lambda i: (i, 0)),
    out_shape=jax.ShapeDtypeStruct((N_VOCAB, AUXW), jnp.int32),
)


def _tc_loss_body(s_ref, o_ref):
    ls = jax.nn.log_sigmoid(s_ref[...])
    o_ref[...] = jnp.reshape(-jnp.sum(ls) / B, (1, 1))


_TC_LOSS = pl.pallas_call(
    _tc_loss_body,
    out_shape=jax.ShapeDtypeStruct((1, 1), jnp.float32),
)


def kernel(x, t, decs, paths, lens):
    t = t.astype(jnp.int32)
    aux = _AUX_BUILD(paths.astype(jnp.int32),
                     lens.astype(jnp.int32).reshape(N_VOCAB // _AUX_BK, 1,
                                                    _AUX_BK))
    scores = _SC_SCORES(x, t, aux, decs)
    loss = _TC_LOSS(scores)
    return loss[0, 0]


# speedup vs baseline: 1.0827x; 1.0377x over previous
import functools

import jax
import jax.numpy as jnp
from jax import lax
from jax.experimental import pallas as pl
from jax.experimental.pallas import tpu as pltpu
from jax.experimental.pallas import tpu_sc as plsc

N_VOCAB = 100000
N_DEC = N_VOCAB - 1
MAX_PATH = 32
D = 512
B = 8192

NC = 2
NS = 16
LANES = 16
NW = NC * NS
TPW = B // NW
NCHUNK = D // LANES
AUXW = 128
SENTINEL = 1e4


def _sc_scores():
    mesh = plsc.VectorSubcoreMesh(core_axis_name="c", subcore_axis_name="s")

    @functools.partial(
        pl.kernel,
        out_type=jax.ShapeDtypeStruct((B, MAX_PATH), jnp.float32),
        mesh=mesh,
        compiler_params=pltpu.CompilerParams(needs_layout_passes=False),
        scratch_types=[
            pltpu.VMEM((TPW,), jnp.int32),
            pltpu.VMEM((TPW, AUXW), jnp.int32),
            pltpu.VMEM((2, 16, D), jnp.float32),
            pltpu.VMEM((2, MAX_PATH, D), jnp.float32),
            pltpu.VMEM((LANES * LANES,), jnp.float32),
            pltpu.VMEM((TPW, MAX_PATH), jnp.float32),
            pltpu.SemaphoreType.DMA,
            pltpu.SemaphoreType.DMA,
            pltpu.SemaphoreType.DMA,
            pltpu.SemaphoreType.DMA,
            pltpu.SemaphoreType.DMA,
            pltpu.SemaphoreType.DMA,
        ],
    )
    def sc_kernel(x_hbm, t_hbm, aux_hbm, decs_hbm,
                  scores_hbm,
                  t_v, aux_v, xrow_v, rows_v, accs_v, sb_v,
                  semr0, semr1, semh0, semh1, semx0, semx1):
        wid = lax.axis_index("s") * NC + lax.axis_index("c")
        base = wid * TPW

        pltpu.sync_copy(t_hbm.at[pl.ds(base, TPW)], t_v)
        for g in range(TPW // 128):
            sl = pl.ds(g * 128, 128)
            pltpu.async_copy(aux_hbm.at[t_v.at[sl]], aux_v.at[sl],
                             semr0).wait()

        iota16 = lax.iota(jnp.int32, LANES)
        sent16 = jnp.full((LANES,), SENTINEL, jnp.float32)
        semr = (semr0, semr1)
        semh = (semh0, semh1)
        semx = (semx0, semx1)

        def lenvec(i):
            return aux_v[i, pl.ds(MAX_PATH, LANES)]

        def fire(i, slot):
            pltpu.async_copy(decs_hbm.at[aux_v.at[i, pl.ds(0, LANES)]],
                             rows_v.at[slot, pl.ds(0, LANES)], semr[slot])

            @pl.when(lenvec(i)[0] > LANES)
            def _():
                pltpu.async_copy(decs_hbm.at[aux_v.at[i, pl.ds(LANES, LANES)]],
                                 rows_v.at[slot, pl.ds(LANES, LANES)],
                                 semh[slot])

        def wait(i, slot):
            pltpu.make_async_copy(decs_hbm.at[aux_v.at[i, pl.ds(0, LANES)]],
                                  rows_v.at[slot, pl.ds(0, LANES)],
                                  semr[slot]).wait()

            @pl.when(lenvec(i)[0] > LANES)
            def _():
                pltpu.make_async_copy(
                    decs_hbm.at[aux_v.at[i, pl.ds(LANES, LANES)]],
                    rows_v.at[slot, pl.ds(LANES, LANES)], semh[slot]).wait()

        def group(i, slot, jg):
            rows = rows_v.at[slot]
            len_bc = lenvec(i)
            zero = jnp.zeros((LANES,), jnp.float32)

            def cb_body(cb, accs):
                out = list(accs)
                for u in range(4):
                    off = (cb * 4 + u) * LANES
                    xc = xrow_v[(i // 16) % 2, i % 16, pl.ds(off, LANES)]
                    for j16 in range(LANES):
                        j = jg * LANES + j16
                        out[j16] = out[j16] + rows[j, pl.ds(off, LANES)] * xc
                return tuple(out)

            accs = lax.fori_loop(0, NCHUNK // 4, cb_body, (zero,) * LANES)
            for j16 in range(LANES):
                accs_v[pl.ds(j16 * LANES, LANES)] = accs[j16]
            row_base = iota16 * LANES
            s0 = plsc.load_gather(accs_v, [row_base])
            s1 = plsc.load_gather(accs_v, [row_base + 1])
            for k in range(2, LANES, 2):
                s0 = s0 + plsc.load_gather(accs_v, [row_base + k])
                s1 = s1 + plsc.load_gather(accs_v, [row_base + k + 1])
            mask = (iota16 + jg * LANES) < len_bc
            s = jnp.where(mask, s0 + s1, SENTINEL)
            sb_v[i, pl.ds(jg * LANES, LANES)] = s

        def compute(i, slot):
            group(i, slot, 0)
            sb_v[i, pl.ds(LANES, LANES)] = sent16

            @pl.when(lenvec(i)[0] > LANES)
            def _():
                group(i, slot, 1)

        def xfire(c):
            pltpu.async_copy(x_hbm.at[pl.ds(base + c * 16, 16)],
                             xrow_v.at[c % 2], semx0)

        def xwait(c):
            pltpu.make_async_copy(x_hbm.at[pl.ds(base + c * 16, 16)],
                                  xrow_v.at[c % 2], semx0).wait()

        xfire(0)
        xwait(0)
        fire(0, 0)

        def pair_body(g, _):
            i0 = 2 * g

            @pl.when((g % 8 == 4) & (i0 // 16 + 1 < TPW // 16))
            def _():
                xfire(i0 // 16 + 1)

            @pl.when((g % 8 == 0) & (g > 0))
            def _():
                xwait(i0 // 16)

            fire(i0 + 1, 1)
            wait(i0, 0)
            compute(i0, 0)

            @pl.when(g < TPW // 2 - 1)
            def _():
                fire(i0 + 2, 0)

            wait(i0 + 1, 1)
            compute(i0 + 1, 1)
            return 0

        lax.fori_loop(0, TPW // 2, pair_body, 0)
        pltpu.sync_copy(sb_v, scores_hbm.at[pl.ds(base, TPW)])

    return sc_kernel


_SC_SCORES = _sc_scores()


_AUX_BK = 4000


def _aux_body(p_ref, l_ref, o_ref):
    p = p_ref[...]
    l = l_ref[0]
    lt = jnp.broadcast_to(jnp.transpose(l, (1, 0)), (_AUX_BK, LANES))
    o_ref[...] = jnp.concatenate(
        [
            p,
            lt,
            jnp.zeros((_AUX_BK, AUXW - MAX_PATH - LANES), jnp.int32),
        ],
        axis=1,
    )


_AUX_BUILD = pl.pallas_call(
    _aux_body,
    grid=(N_VOCAB // _AUX_BK,),
    in_specs=[
        pl.BlockSpec((_AUX_BK, MAX_PATH), lambda i: (i, 0)),
        pl.BlockSpec((1, 1, _AUX_BK), lambda i: (i, 0, 0)),
    ],
    out_specs=pl.BlockSpec((_AUX_BK, AUXW), lambda i: (i, 0)),
    out_shape=jax.ShapeDtypeStruct((N_VOCAB, AUXW), jnp.int32),
)


def _tc_loss_body(s_ref, o_ref):
    ls = jax.nn.log_sigmoid(s_ref[...])
    o_ref[...] = jnp.reshape(-jnp.sum(ls) / B, (1, 1))


_TC_LOSS = pl.pallas_call(
    _tc_loss_body,
    out_shape=jax.ShapeDtypeStruct((1, 1), jnp.float32),
)


def kernel(x, t, decs, paths, lens):
    t = t.astype(jnp.int32)
    aux = _AUX_BUILD(paths.astype(jnp.int32),
                     lens.astype(jnp.int32).reshape(N_VOCAB // _AUX_BK, 1,
                                                    _AUX_BK))
    scores = _SC_SCORES(x, t, aux, decs)
    loss = _TC_LOSS(scores)
    return loss[0, 0]


# speedup vs baseline: 1.1095x; 1.0247x over previous
import functools

import jax
import jax.numpy as jnp
from jax import lax
from jax.experimental import pallas as pl
from jax.experimental.pallas import tpu as pltpu
from jax.experimental.pallas import tpu_sc as plsc

N_VOCAB = 100000
N_DEC = N_VOCAB - 1
MAX_PATH = 32
D = 512
B = 8192

NC = 2
NS = 16
LANES = 16
NW = NC * NS
TPW = B // NW
NCHUNK = D // LANES
AUXW = 128
SENTINEL = 1e4


def _sc_scores():
    mesh = plsc.VectorSubcoreMesh(core_axis_name="c", subcore_axis_name="s")

    @functools.partial(
        pl.kernel,
        out_type=jax.ShapeDtypeStruct((B, MAX_PATH), jnp.float32),
        mesh=mesh,
        compiler_params=pltpu.CompilerParams(needs_layout_passes=False),
        scratch_types=[
            pltpu.VMEM((TPW,), jnp.int32),
            pltpu.VMEM((TPW, AUXW), jnp.int32),
            pltpu.VMEM((2, 16, D), jnp.float32),
            pltpu.VMEM((2, MAX_PATH, D), jnp.float32),
            pltpu.VMEM((LANES * LANES,), jnp.float32),
            pltpu.VMEM((TPW, MAX_PATH), jnp.float32),
            pltpu.SemaphoreType.DMA,
            pltpu.SemaphoreType.DMA,
            pltpu.SemaphoreType.DMA,
            pltpu.SemaphoreType.DMA,
            pltpu.SemaphoreType.DMA,
            pltpu.SemaphoreType.DMA,
        ],
    )
    def sc_kernel(x_hbm, t_hbm, aux_hbm, decs_hbm,
                  scores_hbm,
                  t_v, aux_v, xrow_v, rows_v, accs_v, sb_v,
                  semr0, semr1, semh0, semh1, semx0, semx1):
        wid = lax.axis_index("s") * NC + lax.axis_index("c")
        base = wid * TPW

        pltpu.sync_copy(t_hbm.at[pl.ds(base, TPW)], t_v)
        for g in range(TPW // 128):
            sl = pl.ds(g * 128, 128)
            pltpu.async_copy(aux_hbm.at[t_v.at[sl]], aux_v.at[sl],
                             semr0).wait()

        iota16 = lax.iota(jnp.int32, LANES)
        sent16 = jnp.full((LANES,), SENTINEL, jnp.float32)
        semr = (semr0, semr1)
        semh = (semh0, semh1)
        semx = (semx0, semx1)

        def lenvec(i):
            return aux_v[i, pl.ds(MAX_PATH, LANES)]

        def fire(i, slot):
            pltpu.async_copy(decs_hbm.at[aux_v.at[i, pl.ds(0, LANES)]],
                             rows_v.at[slot, pl.ds(0, LANES)], semr[slot])

            @pl.when(lenvec(i)[0] > LANES)
            def _():
                pltpu.async_copy(decs_hbm.at[aux_v.at[i, pl.ds(LANES, LANES)]],
                                 rows_v.at[slot, pl.ds(LANES, LANES)],
                                 semh[slot])

        def wait(i, slot):
            pltpu.make_async_copy(decs_hbm.at[aux_v.at[i, pl.ds(0, LANES)]],
                                  rows_v.at[slot, pl.ds(0, LANES)],
                                  semr[slot]).wait()

            @pl.when(lenvec(i)[0] > LANES)
            def _():
                pltpu.make_async_copy(
                    decs_hbm.at[aux_v.at[i, pl.ds(LANES, LANES)]],
                    rows_v.at[slot, pl.ds(LANES, LANES)], semh[slot]).wait()

        def group(i, slot, jg):
            rows = rows_v.at[slot]
            len_bc = lenvec(i)
            zero = jnp.zeros((LANES,), jnp.float32)

            def cb_body(cb, accs):
                out = list(accs)
                for u in range(4):
                    off = (cb * 4 + u) * LANES
                    xc = xrow_v[(i // 16) % 2, i % 16, pl.ds(off, LANES)]
                    for j16 in range(LANES):
                        j = jg * LANES + j16
                        out[j16] = out[j16] + rows[j, pl.ds(off, LANES)] * xc
                return tuple(out)

            accs = lax.fori_loop(0, NCHUNK // 4, cb_body, (zero,) * LANES)
            for j16 in range(LANES):
                accs_v[pl.ds(j16 * LANES, LANES)] = accs[j16]
            row_base = iota16 * LANES
            s0 = plsc.load_gather(accs_v, [row_base])
            s1 = plsc.load_gather(accs_v, [row_base + 1])
            for k in range(2, LANES, 2):
                s0 = s0 + plsc.load_gather(accs_v, [row_base + k])
                s1 = s1 + plsc.load_gather(accs_v, [row_base + k + 1])
            mask = (iota16 + jg * LANES) < len_bc
            s = jnp.where(mask, s0 + s1, SENTINEL)
            sb_v[i, pl.ds(jg * LANES, LANES)] = s

        def compute(i, slot):
            group(i, slot, 0)
            sb_v[i, pl.ds(LANES, LANES)] = sent16

            @pl.when(lenvec(i)[0] > LANES)
            def _():
                group(i, slot, 1)

        def xfire(c):
            pltpu.async_copy(x_hbm.at[pl.ds(base + c * 16, 16)],
                             xrow_v.at[c % 2], semx0)

        def xwait(c):
            pltpu.make_async_copy(x_hbm.at[pl.ds(base + c * 16, 16)],
                                  xrow_v.at[c % 2], semx0).wait()

        xfire(0)
        xwait(0)
        fire(0, 0)

        def pair_body(g, _):
            i0 = 2 * g

            @pl.when((g % 8 == 4) & (i0 // 16 + 1 < TPW // 16))
            def _():
                xfire(i0 // 16 + 1)

            @pl.when((g % 8 == 0) & (g > 0))
            def _():
                xwait(i0 // 16)

            fire(i0 + 1, 1)
            wait(i0, 0)
            compute(i0, 0)

            @pl.when(g < TPW // 2 - 1)
            def _():
                fire(i0 + 2, 0)

            wait(i0 + 1, 1)
            compute(i0 + 1, 1)
            return 0

        lax.fori_loop(0, TPW // 2, pair_body, 0)
        pltpu.sync_copy(sb_v, scores_hbm.at[pl.ds(base, TPW)])

    return sc_kernel


_SC_SCORES = _sc_scores()


_AUX_BK = 10000


def _aux_body(p_ref, l_ref, o_ref):
    p = p_ref[...]
    l = l_ref[0]
    lt = jnp.broadcast_to(jnp.transpose(l, (1, 0)), (_AUX_BK, LANES))
    o_ref[...] = jnp.concatenate(
        [
            p,
            lt,
            jnp.zeros((_AUX_BK, AUXW - MAX_PATH - LANES), jnp.int32),
        ],
        axis=1,
    )


_AUX_BUILD = pl.pallas_call(
    _aux_body,
    grid=(N_VOCAB // _AUX_BK,),
    in_specs=[
        pl.BlockSpec((_AUX_BK, MAX_PATH), lambda i: (i, 0)),
        pl.BlockSpec((1, 1, _AUX_BK), lambda i: (i, 0, 0)),
    ],
    out_specs=pl.BlockSpec((_AUX_BK, AUXW), lambda i: (i, 0)),
    out_shape=jax.ShapeDtypeStruct((N_VOCAB, AUXW), jnp.int32),
)


def _tc_loss_body(s_ref, o_ref):
    ls = jax.nn.log_sigmoid(s_ref[...])
    o_ref[...] = jnp.reshape(-jnp.sum(ls) / B, (1, 1))


_TC_LOSS = pl.pallas_call(
    _tc_loss_body,
    out_shape=jax.ShapeDtypeStruct((1, 1), jnp.float32),
)


def kernel(x, t, decs, paths, lens):
    t = t.astype(jnp.int32)
    aux = _AUX_BUILD(paths.astype(jnp.int32),
                     lens.astype(jnp.int32).reshape(N_VOCAB // _AUX_BK, 1,
                                                    _AUX_BK))
    scores = _SC_SCORES(x, t, aux, decs)
    loss = _TC_LOSS(scores)
    return loss[0, 0]
